# baseline (speedup 1.0000x reference)
import functools

import jax
import jax.numpy as jnp
from jax.experimental import pallas as pl
from jax.experimental.pallas import tpu as pltpu

_CHEB_K = 3
_BT = 8
_LS = 128


def _precompute_kernel(nv1_ref, nv2_ref, ne_ref, gbp_ref, ubp_ref,
                       s_ref, bg_ref, bu_ref):
    f32 = jnp.float32
    nv1 = nv1_ref[...]
    nv2 = nv2_ref[...]
    n = nv1.shape[0]

    logits = jnp.maximum(
        jnp.dot(nv1, nv2, preferred_element_type=f32), 0.0)
    m = jnp.max(logits, axis=1, keepdims=True)
    e = jnp.exp(logits - m)
    s1 = e / jnp.sum(e, axis=1, keepdims=True)
    row = jax.lax.broadcasted_iota(jnp.int32, (n, n), 0)
    col = jax.lax.broadcasted_iota(jnp.int32, (n, n), 1)
    eye = (row == col).astype(f32)
    s2 = 2.0 * jnp.dot(s1, s1, preferred_element_type=f32) - eye
    s_ref[0] = s1
    s_ref[1] = s2
    bg_ref[...] = jnp.dot(ne_ref[...], gbp_ref[...], preferred_element_type=f32)
    bu_ref[...] = jnp.dot(ne_ref[...], ubp_ref[...], preferred_element_type=f32)


def _cell_kernel(cheb_k, embed_dim, dim_in, hid,
                 feat_ref, s_ref, ne_ref, wcat_ref, wuh_ref,
                 bg_ref, bu_ref, out_ref):
    f32, bf16 = jnp.float32, jnp.bfloat16
    K, D, Cx, H = cheb_k, embed_dim, dim_in, hid
    npad = s_ref.shape[1]
    rows = feat_ref.shape[1]
    bt = rows // npad

    def mm(a, w):
        return jnp.dot(a, w, preferred_element_type=f32)

    def to_rows(pm, w):
        return jnp.concatenate(
            [pm[:, b * w:(b + 1) * w] for b in range(bt)], axis=0)

    def to_prop(rw, w):
        return jnp.concatenate(
            [rw[b * npad:(b + 1) * npad, :] for b in range(bt)], axis=1)

    def dexpand(rw, w):
        return jnp.concatenate(
            [rw * ne_ref[:, d * _LS:d * _LS + w] for d in range(D)], axis=1)

    feat128 = feat_ref[0]
    st_rows = feat128[:, Cx:Cx + H]
    feat_bf = feat128.astype(bf16)
    featp = to_prop(feat_bf, _LS)

    a_parts = [dexpand(feat_bf, _LS)]
    for k in range(1, K):
        pk = jnp.dot(s_ref[k - 1], featp, preferred_element_type=f32)
        a_parts.append(dexpand(to_rows(pk, _LS).astype(bf16), _LS))
    t_cat = mm(jnp.concatenate(a_parts, axis=1), wcat_ref[...])
    zr = jax.nn.sigmoid((t_cat[:, :2 * H] + bg_ref[...]).astype(bf16))
    z = zr[:, :H]
    r = zr[:, H:].astype(f32)
    t_u = t_cat[:, 2 * H:2 * H + H]

    zs = (z.astype(f32) * st_rows).astype(bf16)
    zsp = to_prop(zs, H)
    a_parts = [dexpand(zs, H)]
    for k in range(1, K):
        pk = jnp.dot(s_ref[k - 1], zsp, preferred_element_type=f32)
        a_parts.append(dexpand(to_rows(pk, H).astype(bf16), H))
    t_u = t_u + mm(jnp.concatenate(a_parts, axis=1), wuh_ref[...])

    hc = jnp.tanh((t_u + bu_ref[...]).astype(bf16)).astype(f32)
    out_ref[0] = r * st_rows + (1.0 - r) * hc


def kernel(x, state, nodevec1, nodevec2,
           gate_weights_pool, gate_bias_pool,
           update_weights_pool, update_bias_pool):
    f32, bf16 = jnp.float32, jnp.bfloat16
    B, N, Cx = x.shape
    H = state.shape[-1]
    D = nodevec1.shape[1]
    K = _CHEB_K
    Ci = Cx + H
    npad = (N + 7) // 8 * 8
    bt = _BT
    while B % bt:
        bt //= 2
    grid_n = B // bt
    rows = bt * npad

    x = x.astype(f32)
    state = state.astype(f32)

    ne = nodevec1 + nodevec2.T

    vmem = pl.BlockSpec(memory_space=pltpu.MemorySpace.VMEM)
    s, bg, bu = pl.pallas_call(
        _precompute_kernel,
        out_shape=(
            jax.ShapeDtypeStruct((K - 1, N, N), f32),
            jax.ShapeDtypeStruct((N, 2 * H), f32),
            jax.ShapeDtypeStruct((N, H), f32),
        ),
        in_specs=[vmem] * 5,
        out_specs=(vmem, vmem, vmem),
    )(nodevec1, nodevec2, ne, gate_bias_pool, update_bias_pool)

    def fold_pad(pool, slot):
        d, kk, c, co = pool.shape
        p = jnp.pad(pool, ((0, 0), (0, 0), (0, slot - c), (0, 0)))
        return jnp.transpose(p, (1, 0, 2, 3)).reshape(kk, d * slot, co)

    xmask = (jnp.arange(Ci) < Cx)[None, None, :, None]
    wcat = jnp.concatenate(
        [fold_pad(gate_weights_pool, _LS),
         fold_pad(update_weights_pool * xmask, _LS)],
        axis=2).reshape(K * D * _LS, 3 * H).astype(bf16)
    wuh = fold_pad(update_weights_pool[:, :, Cx:, :],
                   H).reshape(K * D * H, H).astype(bf16)

    ne_p = jnp.pad(ne, ((0, npad - N), (0, 0)))
    ne_rows = jnp.tile(jnp.repeat(ne_p, _LS, axis=1),
                       (bt, 1)).astype(bf16)
    bg_rows = jnp.tile(jnp.pad(bg, ((0, npad - N), (0, 0))), (bt, 1))
    bu_rows = jnp.tile(jnp.pad(bu, ((0, npad - N), (0, 0))), (bt, 1))
    s_p = jnp.pad(s, ((0, 0), (0, npad - N), (0, npad - N))).astype(bf16)

    featpad = jnp.pad(jnp.concatenate([x, state], axis=-1),
                      ((0, 0), (0, npad - N), (0, _LS - Ci)))
    feat_rows = featpad.reshape(grid_n, rows, _LS)

    out = pl.pallas_call(
        functools.partial(_cell_kernel, K, D, Cx, H),
        out_shape=jax.ShapeDtypeStruct((grid_n, rows, H), f32),
        grid=(grid_n,),
        in_specs=[
            pl.BlockSpec((1, rows, _LS), lambda i: (i, 0, 0)),
            _resident((K - 1, npad, npad)),
            _resident((rows, D * _LS)),
            _resident((K * D * _LS, 3 * H)),
            _resident((K * D * H, H)),
            _resident((rows, 2 * H)),
            _resident((rows, H)),
        ],
        out_specs=pl.BlockSpec((1, rows, H), lambda i: (i, 0, 0)),
        compiler_params=pltpu.CompilerParams(
            dimension_semantics=("parallel",),
            vmem_limit_bytes=100 * 1024 * 1024),
    )(feat_rows, s_p, ne_rows, wcat, wuh, bg_rows, bu_rows)

    return out.reshape(B, npad, H)[:, :N, :]


def _resident(shape):
    return pl.BlockSpec(shape, lambda i, _z=(0,) * len(shape): _z,
                        pipeline_mode=pl.Buffered(1))


# speedup vs baseline: 1.0109x; 1.0109x over previous
import functools

import jax
import jax.numpy as jnp
from jax.experimental import pallas as pl
from jax.experimental.pallas import tpu as pltpu

_CHEB_K = 3
_BT = 16
_LS = 128


def _precompute_kernel(nv1_ref, nv2_ref, ne_ref, gbp_ref, ubp_ref,
                       s_ref, bg_ref, bu_ref):
    f32 = jnp.float32
    nv1 = nv1_ref[...]
    nv2 = nv2_ref[...]
    n = nv1.shape[0]

    logits = jnp.maximum(
        jnp.dot(nv1, nv2, preferred_element_type=f32), 0.0)
    m = jnp.max(logits, axis=1, keepdims=True)
    e = jnp.exp(logits - m)
    s1 = e / jnp.sum(e, axis=1, keepdims=True)
    row = jax.lax.broadcasted_iota(jnp.int32, (n, n), 0)
    col = jax.lax.broadcasted_iota(jnp.int32, (n, n), 1)
    eye = (row == col).astype(f32)
    s2 = 2.0 * jnp.dot(s1, s1, preferred_element_type=f32) - eye
    s_ref[0] = s1
    s_ref[1] = s2
    bg_ref[...] = jnp.dot(ne_ref[...], gbp_ref[...], preferred_element_type=f32)
    bu_ref[...] = jnp.dot(ne_ref[...], ubp_ref[...], preferred_element_type=f32)


def _cell_kernel(cheb_k, embed_dim, dim_in, hid,
                 feat_ref, s_ref, ne_ref, wcat_ref, wuh_ref,
                 bg_ref, bu_ref, out_ref):
    f32, bf16 = jnp.float32, jnp.bfloat16
    K, D, Cx, H = cheb_k, embed_dim, dim_in, hid
    npad = s_ref.shape[1]
    rows = feat_ref.shape[1]
    bt = rows // npad

    def mm(a, w):
        return jnp.dot(a, w, preferred_element_type=f32)

    def to_rows(pm, w):
        return jnp.concatenate(
            [pm[:, b * w:(b + 1) * w] for b in range(bt)], axis=0)

    def to_prop(rw, w):
        return jnp.concatenate(
            [rw[b * npad:(b + 1) * npad, :] for b in range(bt)], axis=1)

    def dexpand(rw, w):
        return jnp.concatenate(
            [rw * ne_ref[:, d * _LS:d * _LS + w] for d in range(D)], axis=1)

    feat128 = feat_ref[0]
    st_rows = feat128[:, Cx:Cx + H]
    feat_bf = feat128.astype(bf16)
    featp = to_prop(feat_bf, _LS)

    a_parts = [dexpand(feat_bf, _LS)]
    for k in range(1, K):
        pk = jnp.dot(s_ref[k - 1], featp, preferred_element_type=f32)
        a_parts.append(dexpand(to_rows(pk, _LS).astype(bf16), _LS))
    t_cat = mm(jnp.concatenate(a_parts, axis=1), wcat_ref[...])
    zr = jax.nn.sigmoid((t_cat[:, :2 * H] + bg_ref[...]).astype(bf16))
    z = zr[:, :H]
    r = zr[:, H:].astype(f32)
    t_u = t_cat[:, 2 * H:2 * H + H]

    zs = (z.astype(f32) * st_rows).astype(bf16)
    zsp = to_prop(zs, H)
    a_parts = [dexpand(zs, H)]
    for k in range(1, K):
        pk = jnp.dot(s_ref[k - 1], zsp, preferred_element_type=f32)
        a_parts.append(dexpand(to_rows(pk, H).astype(bf16), H))
    t_u = t_u + mm(jnp.concatenate(a_parts, axis=1), wuh_ref[...])

    hc = jnp.tanh((t_u + bu_ref[...]).astype(bf16)).astype(f32)
    out_ref[0] = r * st_rows + (1.0 - r) * hc


def kernel(x, state, nodevec1, nodevec2,
           gate_weights_pool, gate_bias_pool,
           update_weights_pool, update_bias_pool):
    f32, bf16 = jnp.float32, jnp.bfloat16
    B, N, Cx = x.shape
    H = state.shape[-1]
    D = nodevec1.shape[1]
    K = _CHEB_K
    Ci = Cx + H
    npad = (N + 7) // 8 * 8
    bt = _BT
    while B % bt:
        bt //= 2
    grid_n = B // bt
    rows = bt * npad

    x = x.astype(f32)
    state = state.astype(f32)

    ne = nodevec1 + nodevec2.T

    vmem = pl.BlockSpec(memory_space=pltpu.MemorySpace.VMEM)
    s, bg, bu = pl.pallas_call(
        _precompute_kernel,
        out_shape=(
            jax.ShapeDtypeStruct((K - 1, N, N), f32),
            jax.ShapeDtypeStruct((N, 2 * H), f32),
            jax.ShapeDtypeStruct((N, H), f32),
        ),
        in_specs=[vmem] * 5,
        out_specs=(vmem, vmem, vmem),
    )(nodevec1, nodevec2, ne, gate_bias_pool, update_bias_pool)

    def fold_pad(pool, slot):
        d, kk, c, co = pool.shape
        p = jnp.pad(pool, ((0, 0), (0, 0), (0, slot - c), (0, 0)))
        return jnp.transpose(p, (1, 0, 2, 3)).reshape(kk, d * slot, co)

    xmask = (jnp.arange(Ci) < Cx)[None, None, :, None]
    wcat = jnp.concatenate(
        [fold_pad(gate_weights_pool, _LS),
         fold_pad(update_weights_pool * xmask, _LS)],
        axis=2).reshape(K * D * _LS, 3 * H).astype(bf16)
    wuh = fold_pad(update_weights_pool[:, :, Cx:, :],
                   H).reshape(K * D * H, H).astype(bf16)

    ne_p = jnp.pad(ne, ((0, npad - N), (0, 0)))
    ne_rows = jnp.tile(jnp.repeat(ne_p, _LS, axis=1),
                       (bt, 1)).astype(bf16)
    bg_rows = jnp.tile(jnp.pad(bg, ((0, npad - N), (0, 0))), (bt, 1))
    bu_rows = jnp.tile(jnp.pad(bu, ((0, npad - N), (0, 0))), (bt, 1))
    s_p = jnp.pad(s, ((0, 0), (0, npad - N), (0, npad - N))).astype(bf16)

    featpad = jnp.pad(jnp.concatenate([x, state], axis=-1),
                      ((0, 0), (0, npad - N), (0, _LS - Ci)))
    feat_rows = featpad.reshape(grid_n, rows, _LS)

    out = pl.pallas_call(
        functools.partial(_cell_kernel, K, D, Cx, H),
        out_shape=jax.ShapeDtypeStruct((grid_n, rows, H), f32),
        grid=(grid_n,),
        in_specs=[
            pl.BlockSpec((1, rows, _LS), lambda i: (i, 0, 0)),
            _resident((K - 1, npad, npad)),
            _resident((rows, D * _LS)),
            _resident((K * D * _LS, 3 * H)),
            _resident((K * D * H, H)),
            _resident((rows, 2 * H)),
            _resident((rows, H)),
        ],
        out_specs=pl.BlockSpec((1, rows, H), lambda i: (i, 0, 0)),
        compiler_params=pltpu.CompilerParams(
            dimension_semantics=("parallel",),
            vmem_limit_bytes=100 * 1024 * 1024),
    )(feat_rows, s_p, ne_rows, wcat, wuh, bg_rows, bu_rows)

    return out.reshape(B, npad, H)[:, :N, :]


def _resident(shape):
    return pl.BlockSpec(shape, lambda i, _z=(0,) * len(shape): _z,
                        pipeline_mode=pl.Buffered(1))


# speedup vs baseline: 1.0174x; 1.0064x over previous
import functools

import jax
import jax.numpy as jnp
from jax.experimental import pallas as pl
from jax.experimental.pallas import tpu as pltpu

_CHEB_K = 3
_BT = 16
_LS = 128


def _precompute_kernel(nv1_ref, nv2_ref, ne_ref, gbp_ref, ubp_ref,
                       s_ref, bg_ref, bu_ref):
    f32 = jnp.float32
    nv1 = nv1_ref[...]
    nv2 = nv2_ref[...]
    n = nv1.shape[0]

    logits = jnp.maximum(
        jnp.dot(nv1, nv2, preferred_element_type=f32), 0.0)
    m = jnp.max(logits, axis=1, keepdims=True)
    e = jnp.exp(logits - m)
    s1 = e / jnp.sum(e, axis=1, keepdims=True)
    row = jax.lax.broadcasted_iota(jnp.int32, (n, n), 0)
    col = jax.lax.broadcasted_iota(jnp.int32, (n, n), 1)
    eye = (row == col).astype(f32)
    s2 = 2.0 * jnp.dot(s1, s1, preferred_element_type=f32) - eye
    s_ref[0] = s1
    s_ref[1] = s2
    bg_ref[...] = jnp.dot(ne_ref[...], gbp_ref[...], preferred_element_type=f32)
    bu_ref[...] = jnp.dot(ne_ref[...], ubp_ref[...], preferred_element_type=f32)


def _cell_kernel(cheb_k, embed_dim, dim_in, hid,
                 feat_ref, s_ref, ne_ref, wcat_ref, wuh_ref,
                 bg_ref, bu_ref, out_ref):
    f32, bf16 = jnp.float32, jnp.bfloat16
    K, D, Cx, H = cheb_k, embed_dim, dim_in, hid
    npad = s_ref.shape[1]
    rows = feat_ref.shape[1]
    bt = rows // npad

    def mm(a, w):
        return jnp.dot(a, w, preferred_element_type=f32)

    def to_rows(pm, w):
        return jnp.concatenate(
            [pm[:, b * w:(b + 1) * w] for b in range(bt)], axis=0)

    def to_prop(rw, w):
        return jnp.concatenate(
            [rw[b * npad:(b + 1) * npad, :] for b in range(bt)], axis=1)

    def dexpand(rw, w):
        return jnp.concatenate(
            [rw * ne_ref[:, d * _LS:d * _LS + w] for d in range(D)], axis=1)

    feat128 = feat_ref[0]
    st_rows = feat128[:, Cx:Cx + H]
    feat_bf = feat128.astype(bf16)
    featp = to_prop(feat_bf, _LS)

    a_parts = [dexpand(feat_bf, _LS)]
    for k in range(1, K):
        pk = jnp.dot(s_ref[k - 1], featp, preferred_element_type=f32)
        a_parts.append(dexpand(to_rows(pk, _LS).astype(bf16), _LS))
    t_cat = mm(jnp.concatenate(a_parts, axis=1), wcat_ref[...])
    zr = jax.nn.sigmoid((t_cat[:, :2 * H] + bg_ref[...]).astype(bf16))
    z = zr[:, :H]
    r = zr[:, H:].astype(f32)
    t_u = t_cat[:, 2 * H:2 * H + H]

    zs = (z.astype(f32) * st_rows).astype(bf16)
    zsp = to_prop(zs, H)
    a_parts = [dexpand(zs, H)]
    for k in range(1, K):
        pk = jnp.dot(s_ref[k - 1], zsp, preferred_element_type=f32)
        a_parts.append(dexpand(to_rows(pk, H).astype(bf16), H))
    t_u = t_u + mm(jnp.concatenate(a_parts, axis=1), wuh_ref[...])

    hc = jnp.tanh((t_u + bu_ref[...]).astype(bf16)).astype(f32)
    hnew = r * st_rows + (1.0 - r) * hc
    nvalid = out_ref.shape[2]
    out_ref[0] = hnew.reshape(bt, npad, H)[:, :nvalid, :]


def kernel(x, state, nodevec1, nodevec2,
           gate_weights_pool, gate_bias_pool,
           update_weights_pool, update_bias_pool):
    f32, bf16 = jnp.float32, jnp.bfloat16
    B, N, Cx = x.shape
    H = state.shape[-1]
    D = nodevec1.shape[1]
    K = _CHEB_K
    Ci = Cx + H
    npad = (N + 7) // 8 * 8
    bt = _BT
    while B % bt:
        bt //= 2
    grid_n = B // bt
    rows = bt * npad

    x = x.astype(f32)
    state = state.astype(f32)

    ne = nodevec1 + nodevec2.T

    vmem = pl.BlockSpec(memory_space=pltpu.MemorySpace.VMEM)
    s, bg, bu = pl.pallas_call(
        _precompute_kernel,
        out_shape=(
            jax.ShapeDtypeStruct((K - 1, N, N), f32),
            jax.ShapeDtypeStruct((N, 2 * H), f32),
            jax.ShapeDtypeStruct((N, H), f32),
        ),
        in_specs=[vmem] * 5,
        out_specs=(vmem, vmem, vmem),
    )(nodevec1, nodevec2, ne, gate_bias_pool, update_bias_pool)

    def fold_pad(pool, slot):
        d, kk, c, co = pool.shape
        p = jnp.pad(pool, ((0, 0), (0, 0), (0, slot - c), (0, 0)))
        return jnp.transpose(p, (1, 0, 2, 3)).reshape(kk, d * slot, co)

    xmask = (jnp.arange(Ci) < Cx)[None, None, :, None]
    wcat = jnp.concatenate(
        [fold_pad(gate_weights_pool, _LS),
         fold_pad(update_weights_pool * xmask, _LS)],
        axis=2).reshape(K * D * _LS, 3 * H).astype(bf16)
    wuh = fold_pad(update_weights_pool[:, :, Cx:, :],
                   H).reshape(K * D * H, H).astype(bf16)

    ne_p = jnp.pad(ne, ((0, npad - N), (0, 0)))
    ne_rows = jnp.tile(jnp.repeat(ne_p, _LS, axis=1),
                       (bt, 1)).astype(bf16)
    bg_rows = jnp.tile(jnp.pad(bg, ((0, npad - N), (0, 0))), (bt, 1))
    bu_rows = jnp.tile(jnp.pad(bu, ((0, npad - N), (0, 0))), (bt, 1))
    s_p = jnp.pad(s, ((0, 0), (0, npad - N), (0, npad - N))).astype(bf16)

    featpad = jnp.pad(jnp.concatenate([x, state], axis=-1),
                      ((0, 0), (0, npad - N), (0, _LS - Ci)))
    feat_rows = featpad.reshape(grid_n, rows, _LS)

    out = pl.pallas_call(
        functools.partial(_cell_kernel, K, D, Cx, H),
        out_shape=jax.ShapeDtypeStruct((grid_n, bt, N, H), f32),
        grid=(grid_n,),
        in_specs=[
            pl.BlockSpec((1, rows, _LS), lambda i: (i, 0, 0)),
            _resident((K - 1, npad, npad)),
            _resident((rows, D * _LS)),
            _resident((K * D * _LS, 3 * H)),
            _resident((K * D * H, H)),
            _resident((rows, 2 * H)),
            _resident((rows, H)),
        ],
        out_specs=pl.BlockSpec((1, bt, N, H), lambda i: (i, 0, 0, 0)),
        compiler_params=pltpu.CompilerParams(
            dimension_semantics=("parallel",),
            vmem_limit_bytes=100 * 1024 * 1024),
    )(feat_rows, s_p, ne_rows, wcat, wuh, bg_rows, bu_rows)

    return out.reshape(B, N, H)


def _resident(shape):
    return pl.BlockSpec(shape, lambda i, _z=(0,) * len(shape): _z,
                        pipeline_mode=pl.Buffered(1))
